# baseline (speedup 1.0000x reference)
import jax
import jax.numpy as jnp
from jax.experimental import pallas as pl

DIM = 1024
CODEVECTOR_DIM = 256
GROUPS = 2
NUM_VARS = 320
GV = GROUPS * NUM_VARS
D_PER_G = CODEVECTOR_DIM // GROUPS

BLOCK_T = 2048


def _vq_kernel(hs_ref, w_ref, b_ref, cb_ref, cv_ref, dist_ref):
    hs = hs_ref[...]
    w = w_ref[...]
    lt = jax.lax.dot_general(
        w, hs, (((0,), (1,)), ((), ())), preferred_element_type=jnp.float32)
    lt = lt + b_ref[...]
    bt = lt.shape[1]
    iota = jax.lax.broadcasted_iota(jnp.int32, (NUM_VARS, bt), 0)
    cvs = []
    for g in range(GROUPS):
        lg = lt[g * NUM_VARS:(g + 1) * NUM_VARS, :]
        m = jnp.max(lg, axis=0, keepdims=True)
        mask = lg == m
        mi = jnp.min(jnp.where(mask, iota, NUM_VARS), axis=0, keepdims=True)
        oht = (iota == mi).astype(jnp.float32)
        dist_ref[g * NUM_VARS:(g + 1) * NUM_VARS, :] = oht
        cb_g = cb_ref[g * NUM_VARS:(g + 1) * NUM_VARS, :]
        cvs.append(jax.lax.dot_general(
            oht, cb_g, (((0,), (0,)), ((), ())),
            preferred_element_type=jnp.float32))
    cv_ref[...] = jnp.concatenate(cvs, axis=1)


def kernel(hidden_states, codevectors, w_proj, b_proj):
    B, S, H = hidden_states.shape
    T = B * S
    hs = hidden_states.reshape(T, H)
    cb = codevectors.reshape(GV, D_PER_G)
    b2 = b_proj.reshape(GV, 1)

    grid = (T // BLOCK_T,)
    cv, dist_t = pl.pallas_call(
        _vq_kernel,
        grid=grid,
        in_specs=[
            pl.BlockSpec((BLOCK_T, H), lambda i: (i, 0)),
            pl.BlockSpec((H, GV), lambda i: (0, 0)),
            pl.BlockSpec((GV, 1), lambda i: (0, 0)),
            pl.BlockSpec((GV, D_PER_G), lambda i: (0, 0)),
        ],
        out_specs=[
            pl.BlockSpec((BLOCK_T, CODEVECTOR_DIM), lambda i: (i, 0)),
            pl.BlockSpec((GV, BLOCK_T), lambda i: (0, i)),
        ],
        out_shape=[
            jax.ShapeDtypeStruct((T, CODEVECTOR_DIM), jnp.float32),
            jax.ShapeDtypeStruct((GV, T), jnp.float32),
        ],
    )(hs, w_proj, b2, cb)
    dist = dist_t.T.reshape(T, GROUPS, NUM_VARS)
    return cv.reshape(B, S, CODEVECTOR_DIM), dist


# speedup vs baseline: 1.0466x; 1.0466x over previous
import jax
import jax.numpy as jnp
from jax.experimental import pallas as pl
from jax.experimental.pallas import tpu as pltpu

DIM = 1024
CODEVECTOR_DIM = 256
GROUPS = 2
NUM_VARS = 320
GV = GROUPS * NUM_VARS
D_PER_G = CODEVECTOR_DIM // GROUPS

BLOCK_T = 1024


def _vq_kernel(hs_ref, w_ref, b_ref, cb_ref, cv_ref, dist_ref):
    hs = hs_ref[...]
    w = w_ref[...]
    lt = jax.lax.dot_general(
        w, hs, (((0,), (1,)), ((), ())), preferred_element_type=jnp.float32)
    lt = lt + b_ref[...]
    bt = lt.shape[1]
    iota = jax.lax.broadcasted_iota(jnp.int32, (NUM_VARS, bt), 0)
    cvs = []
    for g in range(GROUPS):
        lg = lt[g * NUM_VARS:(g + 1) * NUM_VARS, :]
        m = jnp.max(lg, axis=0, keepdims=True)
        mask = lg == m
        mi = jnp.min(jnp.where(mask, iota, NUM_VARS), axis=0, keepdims=True)
        oht = (iota == mi).astype(jnp.float32)
        dist_ref[g * NUM_VARS:(g + 1) * NUM_VARS, :] = oht
        cb_g = cb_ref[g * NUM_VARS:(g + 1) * NUM_VARS, :]
        cvs.append(jax.lax.dot_general(
            oht, cb_g, (((0,), (0,)), ((), ())),
            preferred_element_type=jnp.float32))
    cv_ref[...] = jnp.concatenate(cvs, axis=1)


def kernel(hidden_states, codevectors, w_proj, b_proj):
    B, S, H = hidden_states.shape
    T = B * S
    hs = hidden_states.reshape(T, H)
    cb = codevectors.reshape(GV, D_PER_G)
    b2 = b_proj.reshape(GV, 1)

    grid = (T // BLOCK_T,)
    cv, dist_t = pl.pallas_call(
        _vq_kernel,
        grid=grid,
        in_specs=[
            pl.BlockSpec((BLOCK_T, H), lambda i: (i, 0)),
            pl.BlockSpec((H, GV), lambda i: (0, 0)),
            pl.BlockSpec((GV, 1), lambda i: (0, 0)),
            pl.BlockSpec((GV, D_PER_G), lambda i: (0, 0)),
        ],
        out_specs=[
            pl.BlockSpec((BLOCK_T, CODEVECTOR_DIM), lambda i: (i, 0)),
            pl.BlockSpec((GV, BLOCK_T), lambda i: (0, i)),
        ],
        out_shape=[
            jax.ShapeDtypeStruct((T, CODEVECTOR_DIM), jnp.float32),
            jax.ShapeDtypeStruct((GV, T), jnp.float32),
        ],
    )(hs, w_proj, b2, cb)
    dist = dist_t.T.reshape(T, GROUPS, NUM_VARS)
    return cv.reshape(B, S, CODEVECTOR_DIM), dist


# speedup vs baseline: 1.0527x; 1.0058x over previous
import jax
import jax.numpy as jnp
from jax.experimental import pallas as pl
from jax.experimental.pallas import tpu as pltpu

DIM = 1024
CODEVECTOR_DIM = 256
GROUPS = 2
NUM_VARS = 320
GV = GROUPS * NUM_VARS
D_PER_G = CODEVECTOR_DIM // GROUPS

BLOCK_T = 1024


def _vq_kernel(hs_ref, w_ref, b_ref, cb_ref, cv_ref, dist_ref):
    hs = hs_ref[...]
    w = w_ref[...]
    lt = jax.lax.dot_general(
        w, hs, (((0,), (1,)), ((), ())), preferred_element_type=jnp.float32)
    lt = lt + b_ref[...]
    bt = lt.shape[1]
    iota = jax.lax.broadcasted_iota(jnp.int32, (NUM_VARS, bt), 0)
    cvs = []
    for g in range(GROUPS):
        lg = lt[g * NUM_VARS:(g + 1) * NUM_VARS, :]
        m = jnp.max(lg, axis=0, keepdims=True)
        mask = lg == m
        mi = jnp.min(jnp.where(mask, iota, NUM_VARS), axis=0, keepdims=True)
        oht = (iota == mi).astype(jnp.float32)
        dist_ref[g * NUM_VARS:(g + 1) * NUM_VARS, :] = oht
        cb_g = cb_ref[g * NUM_VARS:(g + 1) * NUM_VARS, :]
        cvs.append(jax.lax.dot_general(
            oht, cb_g, (((0,), (0,)), ((), ())),
            preferred_element_type=jnp.float32))
    cv_ref[...] = jnp.concatenate(cvs, axis=1)


def kernel(hidden_states, codevectors, w_proj, b_proj):
    B, S, H = hidden_states.shape
    T = B * S
    hs = hidden_states.reshape(T, H)
    cb = codevectors.reshape(GV, D_PER_G)
    b2 = b_proj.reshape(GV, 1)

    grid = (T // BLOCK_T,)
    cv, dist_t = pl.pallas_call(
        _vq_kernel,
        grid=grid,
        in_specs=[
            pl.BlockSpec((BLOCK_T, H), lambda i: (i, 0)),
            pl.BlockSpec((H, GV), lambda i: (0, 0)),
            pl.BlockSpec((GV, 1), lambda i: (0, 0)),
            pl.BlockSpec((GV, D_PER_G), lambda i: (0, 0)),
        ],
        out_specs=[
            pl.BlockSpec((BLOCK_T, CODEVECTOR_DIM), lambda i: (i, 0)),
            pl.BlockSpec((GV, BLOCK_T), lambda i: (0, i)),
        ],
        out_shape=[
            jax.ShapeDtypeStruct((T, CODEVECTOR_DIM), jnp.float32),
            jax.ShapeDtypeStruct((GV, T), jnp.float32),
        ],
        compiler_params=pltpu.CompilerParams(
            dimension_semantics=("parallel",)),
    )(hs, w_proj, b2, cb)
    dist = dist_t.T.reshape(T, GROUPS, NUM_VARS)
    return cv.reshape(B, S, CODEVECTOR_DIM), dist
